# Initial kernel scaffold; baseline (speedup 1.0000x reference)
import jax
import jax.numpy as jnp
from jax.experimental import pallas as pl


def kernel(x, W, b):
    raise NotImplementedError("write your pallas kernel here")


# speedup vs baseline: 6.1191x; 6.1191x over previous
import functools

import jax
import jax.numpy as jnp
from jax import lax
from jax.experimental import pallas as pl
from jax.experimental.pallas import tpu as pltpu

N_TOKENS = 32768
EMB_DIM = 768
NUM_EXPERTS = 64
BEST_K = 8
BLOCK = 512

_NEG_INF = float("-inf")


def _router_body(x_ref, wt_ref, b_ref, router_ref, idx_ref):
    x_blk = x_ref[...]
    wt = wt_ref[...]
    logits = jnp.dot(x_blk, wt, preferred_element_type=jnp.float32)
    logits = logits + b_ref[...]

    col = lax.broadcasted_iota(jnp.int32, logits.shape, 1)
    masked = logits
    selected = jnp.zeros(logits.shape, dtype=jnp.bool_)
    vals = []
    idxs = []
    for _ in range(BEST_K):
        m = jnp.max(masked, axis=1, keepdims=True)
        idx = jnp.min(
            jnp.where(masked == m, col, NUM_EXPERTS), axis=1, keepdims=True
        )
        onehot = col == idx
        selected = jnp.logical_or(selected, onehot)
        masked = jnp.where(onehot, _NEG_INF, masked)
        vals.append(m)
        idxs.append(idx)

    top1 = vals[0]
    e = jnp.where(selected, jnp.exp(logits - top1), 0.0)
    denom = jnp.sum(e, axis=1, keepdims=True)
    router_ref[...] = e / denom
    idx_ref[...] = jnp.concatenate(idxs, axis=1)


@jax.jit
def kernel(x, W, b):
    wt = W.T
    b2 = b.reshape(1, NUM_EXPERTS)
    grid = (N_TOKENS // BLOCK,)
    router, idxs = pl.pallas_call(
        _router_body,
        grid=grid,
        in_specs=[
            pl.BlockSpec((BLOCK, EMB_DIM), lambda i: (i, 0)),
            pl.BlockSpec((EMB_DIM, NUM_EXPERTS), lambda i: (0, 0)),
            pl.BlockSpec((1, NUM_EXPERTS), lambda i: (0, 0)),
        ],
        out_specs=[
            pl.BlockSpec((BLOCK, NUM_EXPERTS), lambda i: (i, 0)),
            pl.BlockSpec((BLOCK, BEST_K), lambda i: (i, 0)),
        ],
        out_shape=[
            jax.ShapeDtypeStruct((N_TOKENS, NUM_EXPERTS), jnp.float32),
            jax.ShapeDtypeStruct((N_TOKENS, BEST_K), jnp.int32),
        ],
        compiler_params=pltpu.CompilerParams(
            dimension_semantics=("arbitrary",),
        ),
    )(x, wt, b2)
    return (router, idxs)


# speedup vs baseline: 8.2031x; 1.3406x over previous
import functools

import jax
import jax.numpy as jnp
from jax import lax
from jax.experimental import pallas as pl
from jax.experimental.pallas import tpu as pltpu

N_TOKENS = 32768
EMB_DIM = 768
NUM_EXPERTS = 64
BEST_K = 8
BLOCK = 512

_NEG_INF = float("-inf")


def _router_body(x_ref, wt_ref, b_ref, router_ref, idx_ref):
    x_blk = x_ref[...]
    wt = wt_ref[...]
    logits = jnp.dot(x_blk, wt, preferred_element_type=jnp.float32)
    logits = logits + b_ref[...]

    colf = lax.broadcasted_iota(jnp.int32, logits.shape, 1).astype(jnp.float32)
    masked = logits
    selected = jnp.zeros(logits.shape, dtype=jnp.bool_)
    vals = []
    idxs = []
    for _ in range(BEST_K):
        m = jnp.max(masked, axis=1, keepdims=True)
        idxf = jnp.min(
            jnp.where(masked == m, colf, float(NUM_EXPERTS)),
            axis=1,
            keepdims=True,
        )
        onehot = colf == idxf
        selected = jnp.logical_or(selected, onehot)
        masked = jnp.where(onehot, _NEG_INF, masked)
        vals.append(m)
        idxs.append(idxf.astype(jnp.int32))

    top1 = vals[0]
    e = jnp.where(selected, jnp.exp(logits - top1), 0.0)
    denom = jnp.sum(e, axis=1, keepdims=True)
    router_ref[...] = e / denom
    idx_ref[...] = jnp.concatenate(idxs, axis=1)


@jax.jit
def kernel(x, W, b):
    wt = W.T
    b2 = b.reshape(1, NUM_EXPERTS)
    grid = (N_TOKENS // BLOCK,)
    router, idxs = pl.pallas_call(
        _router_body,
        grid=grid,
        in_specs=[
            pl.BlockSpec((BLOCK, EMB_DIM), lambda i: (i, 0)),
            pl.BlockSpec((EMB_DIM, NUM_EXPERTS), lambda i: (0, 0)),
            pl.BlockSpec((1, NUM_EXPERTS), lambda i: (0, 0)),
        ],
        out_specs=[
            pl.BlockSpec((BLOCK, NUM_EXPERTS), lambda i: (i, 0)),
            pl.BlockSpec((BLOCK, BEST_K), lambda i: (i, 0)),
        ],
        out_shape=[
            jax.ShapeDtypeStruct((N_TOKENS, NUM_EXPERTS), jnp.float32),
            jax.ShapeDtypeStruct((N_TOKENS, BEST_K), jnp.int32),
        ],
        compiler_params=pltpu.CompilerParams(
            dimension_semantics=("arbitrary",),
        ),
    )(x, wt, b2)
    return (router, idxs)


# speedup vs baseline: 10.8613x; 1.3240x over previous
import functools

import jax
import jax.numpy as jnp
from jax import lax
from jax.experimental import pallas as pl
from jax.experimental.pallas import tpu as pltpu

N_TOKENS = 32768
EMB_DIM = 768
NUM_EXPERTS = 64
BEST_K = 8
BLOCK = 512

_NEG_INF = float("-inf")


def _router_body(x_ref, w_ref, b_ref, router_ref, idx_ref):
    x_blk = x_ref[...]
    w = w_ref[...]
    logits = lax.dot_general(
        w, x_blk, (((1,), (1,)), ((), ())),
        preferred_element_type=jnp.float32,
    )
    logits = logits + b_ref[...]

    rowf = lax.broadcasted_iota(jnp.int32, logits.shape, 0).astype(jnp.float32)
    masked = logits
    vals = []
    idxs = []
    for _ in range(BEST_K):
        m = jnp.max(masked, axis=0, keepdims=True)
        idxf = jnp.min(
            jnp.where(masked == m, rowf, float(NUM_EXPERTS)),
            axis=0,
            keepdims=True,
        )
        onehot = rowf == idxf
        masked = jnp.where(onehot, _NEG_INF, masked)
        vals.append(m)
        idxs.append(idxf)

    top1 = vals[0]
    e = jnp.where(masked == _NEG_INF, jnp.exp(logits - top1), 0.0)
    denom = jnp.sum(e, axis=0, keepdims=True)
    router_ref[...] = (e / denom).T
    idxf8 = jnp.concatenate(idxs, axis=0)
    idx_ref[...] = idxf8.astype(jnp.int32).T


@jax.jit
def kernel(x, W, b):
    b2 = b.reshape(NUM_EXPERTS, 1)
    grid = (N_TOKENS // BLOCK,)
    router, idxs = pl.pallas_call(
        _router_body,
        grid=grid,
        in_specs=[
            pl.BlockSpec((BLOCK, EMB_DIM), lambda i: (i, 0)),
            pl.BlockSpec((NUM_EXPERTS, EMB_DIM), lambda i: (0, 0)),
            pl.BlockSpec((NUM_EXPERTS, 1), lambda i: (0, 0)),
        ],
        out_specs=[
            pl.BlockSpec((BLOCK, NUM_EXPERTS), lambda i: (i, 0)),
            pl.BlockSpec((BLOCK, BEST_K), lambda i: (i, 0)),
        ],
        out_shape=[
            jax.ShapeDtypeStruct((N_TOKENS, NUM_EXPERTS), jnp.float32),
            jax.ShapeDtypeStruct((N_TOKENS, BEST_K), jnp.int32),
        ],
        compiler_params=pltpu.CompilerParams(
            dimension_semantics=("arbitrary",),
        ),
    )(x, W, b2)
    return (router, idxs)


# speedup vs baseline: 13.7857x; 1.2693x over previous
import functools

import jax
import jax.numpy as jnp
from jax import lax
from jax.experimental import pallas as pl
from jax.experimental.pallas import tpu as pltpu

N_TOKENS = 32768
EMB_DIM = 768
NUM_EXPERTS = 64
BEST_K = 8
BLOCK = 1024

_NEG_INF = float("-inf")


def _router_body(x_ref, w_ref, b_ref, router_ref, idx_ref):
    x_blk = x_ref[...]
    w = w_ref[...]
    logits = lax.dot_general(
        w, x_blk, (((1,), (1,)), ((), ())),
        preferred_element_type=jnp.float32,
    )
    logits = logits + b_ref[...]

    rowf = lax.broadcasted_iota(jnp.int32, logits.shape, 0).astype(jnp.float32)
    masked = logits
    vals = []
    idxs = []
    for _ in range(BEST_K):
        m = jnp.max(masked, axis=0, keepdims=True)
        idxf = jnp.min(
            jnp.where(masked == m, rowf, float(NUM_EXPERTS)),
            axis=0,
            keepdims=True,
        )
        onehot = rowf == idxf
        masked = jnp.where(onehot, _NEG_INF, masked)
        vals.append(m)
        idxs.append(idxf)

    top1 = vals[0]
    e = jnp.where(masked == _NEG_INF, jnp.exp(logits - top1), 0.0)
    denom = jnp.sum(e, axis=0, keepdims=True)
    router_ref[...] = (e / denom).T
    idxf8 = jnp.concatenate(idxs, axis=0)
    idx_ref[...] = idxf8.astype(jnp.int32).T


@jax.jit
def kernel(x, W, b):
    b2 = b.reshape(NUM_EXPERTS, 1)
    grid = (N_TOKENS // BLOCK,)
    router, idxs = pl.pallas_call(
        _router_body,
        grid=grid,
        in_specs=[
            pl.BlockSpec((BLOCK, EMB_DIM), lambda i: (i, 0)),
            pl.BlockSpec((NUM_EXPERTS, EMB_DIM), lambda i: (0, 0)),
            pl.BlockSpec((NUM_EXPERTS, 1), lambda i: (0, 0)),
        ],
        out_specs=[
            pl.BlockSpec((BLOCK, NUM_EXPERTS), lambda i: (i, 0)),
            pl.BlockSpec((BLOCK, BEST_K), lambda i: (i, 0)),
        ],
        out_shape=[
            jax.ShapeDtypeStruct((N_TOKENS, NUM_EXPERTS), jnp.float32),
            jax.ShapeDtypeStruct((N_TOKENS, BEST_K), jnp.int32),
        ],
        compiler_params=pltpu.CompilerParams(
            dimension_semantics=("arbitrary",),
        ),
    )(x, W, b2)
    return (router, idxs)


# speedup vs baseline: 15.9900x; 1.1599x over previous
import functools

import jax
import jax.numpy as jnp
from jax import lax
from jax.experimental import pallas as pl
from jax.experimental.pallas import tpu as pltpu

N_TOKENS = 32768
EMB_DIM = 768
NUM_EXPERTS = 64
BEST_K = 8
BLOCK = 2048

_NEG_INF = float("-inf")


def _router_body(x_ref, w_ref, b_ref, router_ref, idx_ref):
    x_blk = x_ref[...]
    w = w_ref[...]
    logits = lax.dot_general(
        w, x_blk, (((1,), (1,)), ((), ())),
        preferred_element_type=jnp.float32,
    )
    logits = logits + b_ref[...]

    rowf = lax.broadcasted_iota(jnp.int32, logits.shape, 0).astype(jnp.float32)
    masked = logits
    vals = []
    idxs = []
    for _ in range(BEST_K):
        m = jnp.max(masked, axis=0, keepdims=True)
        idxf = jnp.min(
            jnp.where(masked == m, rowf, float(NUM_EXPERTS)),
            axis=0,
            keepdims=True,
        )
        onehot = rowf == idxf
        masked = jnp.where(onehot, _NEG_INF, masked)
        vals.append(m)
        idxs.append(idxf)

    top1 = vals[0]
    e = jnp.where(masked == _NEG_INF, jnp.exp(logits - top1), 0.0)
    denom = jnp.sum(e, axis=0, keepdims=True)
    router_ref[...] = (e / denom).T
    idxf8 = jnp.concatenate(idxs, axis=0)
    idx_ref[...] = idxf8.astype(jnp.int32).T


@jax.jit
def kernel(x, W, b):
    b2 = b.reshape(NUM_EXPERTS, 1)
    grid = (N_TOKENS // BLOCK,)
    router, idxs = pl.pallas_call(
        _router_body,
        grid=grid,
        in_specs=[
            pl.BlockSpec((BLOCK, EMB_DIM), lambda i: (i, 0)),
            pl.BlockSpec((NUM_EXPERTS, EMB_DIM), lambda i: (0, 0)),
            pl.BlockSpec((NUM_EXPERTS, 1), lambda i: (0, 0)),
        ],
        out_specs=[
            pl.BlockSpec((BLOCK, NUM_EXPERTS), lambda i: (i, 0)),
            pl.BlockSpec((BLOCK, BEST_K), lambda i: (i, 0)),
        ],
        out_shape=[
            jax.ShapeDtypeStruct((N_TOKENS, NUM_EXPERTS), jnp.float32),
            jax.ShapeDtypeStruct((N_TOKENS, BEST_K), jnp.int32),
        ],
        compiler_params=pltpu.CompilerParams(
            dimension_semantics=("arbitrary",),
        ),
    )(x, W, b2)
    return (router, idxs)


# speedup vs baseline: 16.6067x; 1.0386x over previous
import functools

import jax
import jax.numpy as jnp
from jax import lax
from jax.experimental import pallas as pl
from jax.experimental.pallas import tpu as pltpu

N_TOKENS = 32768
EMB_DIM = 768
NUM_EXPERTS = 64
BEST_K = 8
BLOCK = 4096

_NEG_INF = float("-inf")


def _router_body(x_ref, w_ref, b_ref, router_ref, idx_ref):
    x_blk = x_ref[...]
    w = w_ref[...]
    logits = lax.dot_general(
        w, x_blk, (((1,), (1,)), ((), ())),
        preferred_element_type=jnp.float32,
    )
    logits = logits + b_ref[...]

    rowf = lax.broadcasted_iota(jnp.int32, logits.shape, 0).astype(jnp.float32)
    masked = logits
    vals = []
    idxs = []
    for _ in range(BEST_K):
        m = jnp.max(masked, axis=0, keepdims=True)
        idxf = jnp.min(
            jnp.where(masked == m, rowf, float(NUM_EXPERTS)),
            axis=0,
            keepdims=True,
        )
        onehot = rowf == idxf
        masked = jnp.where(onehot, _NEG_INF, masked)
        vals.append(m)
        idxs.append(idxf)

    top1 = vals[0]
    e = jnp.where(masked == _NEG_INF, jnp.exp(logits - top1), 0.0)
    denom = jnp.sum(e, axis=0, keepdims=True)
    router_ref[...] = (e / denom).T
    idxf8 = jnp.concatenate(idxs, axis=0)
    idx_ref[...] = idxf8.astype(jnp.int32).T


@jax.jit
def kernel(x, W, b):
    b2 = b.reshape(NUM_EXPERTS, 1)
    grid = (N_TOKENS // BLOCK,)
    router, idxs = pl.pallas_call(
        _router_body,
        grid=grid,
        in_specs=[
            pl.BlockSpec((BLOCK, EMB_DIM), lambda i: (i, 0)),
            pl.BlockSpec((NUM_EXPERTS, EMB_DIM), lambda i: (0, 0)),
            pl.BlockSpec((NUM_EXPERTS, 1), lambda i: (0, 0)),
        ],
        out_specs=[
            pl.BlockSpec((BLOCK, NUM_EXPERTS), lambda i: (i, 0)),
            pl.BlockSpec((BLOCK, BEST_K), lambda i: (i, 0)),
        ],
        out_shape=[
            jax.ShapeDtypeStruct((N_TOKENS, NUM_EXPERTS), jnp.float32),
            jax.ShapeDtypeStruct((N_TOKENS, BEST_K), jnp.int32),
        ],
        compiler_params=pltpu.CompilerParams(
            dimension_semantics=("arbitrary",),
        ),
    )(x, W, b2)
    return (router, idxs)
